# Initial kernel scaffold; baseline (speedup 1.0000x reference)
import jax
import jax.numpy as jnp
from jax.experimental import pallas as pl


def kernel(x, weight, bias, gamma, beta):
    raise NotImplementedError("write your pallas kernel here")


# speedup vs baseline: 1.2494x; 1.2494x over previous
import math
from functools import partial

import jax
import jax.numpy as jnp
from jax.experimental import pallas as pl
from jax.experimental.pallas import tpu as pltpu

_BN_EPS = 1e-5
_SLOPE = 0.1
_VMEM_LIMIT = 64 * 1024 * 1024


def _pick_tile(m):
    for t in (2048, 1024, 512, 256, 128, 64, 32, 16, 8):
        if m % t == 0:
            return t
    return m


def _zstats_kernel(x_ref, w_ref, z_ref, sum_ref, sq_ref):
    xb = x_ref[...].astype(jnp.bfloat16)
    z = jnp.dot(xb, w_ref[...], preferred_element_type=jnp.float32)
    z_ref[...] = z.astype(jnp.bfloat16)
    sum_ref[...] = jnp.sum(z, axis=0)[None, None, :]
    sq_ref[...] = jnp.sum(z * z, axis=0)[None, None, :]


def _finish_kernel(z_ref, sum_ref, sq_ref, g_ref, b_ref, o_ref, *, m):
    s1 = jnp.sum(sum_ref[...], axis=0)
    s2 = jnp.sum(sq_ref[...], axis=0)
    inv_m = 1.0 / m
    mean = s1 * inv_m
    var = jnp.maximum(s2 * inv_m - mean * mean, 0.0)
    scale = g_ref[...] * jax.lax.rsqrt(var + _BN_EPS)
    shift = b_ref[...] - mean * scale
    y = z_ref[...].astype(jnp.float32) * scale + shift
    o_ref[...] = jnp.where(y > 0, y, _SLOPE * y)


@jax.jit
def _run(x, weight, gamma, beta):
    B, N, in_dim = x.shape
    out_dim = weight.shape[0]
    M = B * N
    x2 = x.reshape(M, in_dim)
    wt = weight.T.astype(jnp.bfloat16)

    tm = _pick_tile(M)
    n_t = M // tm
    f32 = jnp.float32

    z, psum, psq = pl.pallas_call(
        _zstats_kernel,
        out_shape=(jax.ShapeDtypeStruct((M, out_dim), jnp.bfloat16),
                   jax.ShapeDtypeStruct((n_t, 1, out_dim), f32),
                   jax.ShapeDtypeStruct((n_t, 1, out_dim), f32)),
        grid=(n_t,),
        in_specs=[pl.BlockSpec((tm, in_dim), lambda i: (i, 0)),
                  pl.BlockSpec((in_dim, out_dim), lambda i: (0, 0))],
        out_specs=(pl.BlockSpec((tm, out_dim), lambda i: (i, 0)),
                   pl.BlockSpec((1, 1, out_dim), lambda i: (i, 0, 0)),
                   pl.BlockSpec((1, 1, out_dim), lambda i: (i, 0, 0))),
        compiler_params=pltpu.CompilerParams(
            dimension_semantics=("parallel",),
            vmem_limit_bytes=_VMEM_LIMIT),
    )(x2, wt)

    out = pl.pallas_call(
        partial(_finish_kernel, m=M),
        out_shape=jax.ShapeDtypeStruct((M, out_dim), x.dtype),
        grid=(n_t,),
        in_specs=[pl.BlockSpec((tm, out_dim), lambda i: (i, 0)),
                  pl.BlockSpec((n_t, 1, out_dim), lambda i: (0, 0, 0)),
                  pl.BlockSpec((n_t, 1, out_dim), lambda i: (0, 0, 0)),
                  pl.BlockSpec((1, out_dim), lambda i: (0, 0)),
                  pl.BlockSpec((1, out_dim), lambda i: (0, 0))],
        out_specs=pl.BlockSpec((tm, out_dim), lambda i: (i, 0)),
        compiler_params=pltpu.CompilerParams(
            dimension_semantics=("parallel",),
            vmem_limit_bytes=_VMEM_LIMIT),
    )(z, psum, psq, gamma.reshape(1, out_dim).astype(f32),
      beta.reshape(1, out_dim).astype(f32))

    return out.reshape(B, N, out_dim)


def kernel(x, weight, bias, gamma, beta):
    del bias
    return _run(x, weight, gamma, beta)


# speedup vs baseline: 1.6878x; 1.3509x over previous
import math
from functools import partial

import jax
import jax.numpy as jnp
from jax.experimental import pallas as pl
from jax.experimental.pallas import tpu as pltpu

_BN_EPS = 1e-5
_SLOPE = 0.1
_VMEM_LIMIT = 100 * 1024 * 1024


def _pick_tile(m):
    for t in (2048, 1024, 512, 256, 128, 64, 32, 16, 8):
        if m % t == 0:
            return t
    return m


def _fused_kernel(x_ref, w_ref, g_ref, b_ref, o_ref,
                  z_ref, sum_ref, sq_ref, scale_ref, shift_ref, *, tm, m):
    p = pl.program_id(0)
    i = pl.program_id(1)

    @pl.when(p == 0)
    def _compute():
        @pl.when(i == 0)
        def _init():
            sum_ref[...] = jnp.zeros_like(sum_ref)
            sq_ref[...] = jnp.zeros_like(sq_ref)

        xb = x_ref[...].astype(jnp.bfloat16)
        z = jnp.dot(xb, w_ref[...], preferred_element_type=jnp.float32)
        z_ref[pl.ds(i * tm, tm), :] = z.astype(jnp.bfloat16)
        sum_ref[...] += jnp.sum(z, axis=0, keepdims=True)
        sq_ref[...] += jnp.sum(z * z, axis=0, keepdims=True)

    @pl.when(p == 1)
    def _normalize():
        @pl.when(i == 0)
        def _fold_stats():
            inv_m = 1.0 / m
            mean = sum_ref[...] * inv_m
            var = jnp.maximum(sq_ref[...] * inv_m - mean * mean, 0.0)
            scale_ref[...] = g_ref[...] * jax.lax.rsqrt(var + _BN_EPS)
            shift_ref[...] = b_ref[...] - mean * scale_ref[...]

        zt = z_ref[pl.ds(i * tm, tm), :].astype(jnp.float32)
        y = zt * scale_ref[...] + shift_ref[...]
        o_ref[...] = jnp.where(y > 0, y, _SLOPE * y)


@jax.jit
def _run(x, weight, gamma, beta):
    B, N, in_dim = x.shape
    out_dim = weight.shape[0]
    M = B * N
    x2 = x.reshape(M, in_dim)
    wt = weight.T.astype(jnp.bfloat16)

    tm = _pick_tile(M)
    n_t = M // tm
    f32 = jnp.float32

    out = pl.pallas_call(
        partial(_fused_kernel, tm=tm, m=M),
        out_shape=jax.ShapeDtypeStruct((M, out_dim), x.dtype),
        grid=(2, n_t),
        in_specs=[pl.BlockSpec((tm, in_dim), lambda p, i: ((1 - p) * i, 0)),
                  pl.BlockSpec((in_dim, out_dim), lambda p, i: (0, 0)),
                  pl.BlockSpec((1, out_dim), lambda p, i: (0, 0)),
                  pl.BlockSpec((1, out_dim), lambda p, i: (0, 0))],
        out_specs=pl.BlockSpec((tm, out_dim), lambda p, i: (p * i, 0)),
        scratch_shapes=[pltpu.VMEM((M, out_dim), jnp.bfloat16),
                        pltpu.VMEM((1, out_dim), f32),
                        pltpu.VMEM((1, out_dim), f32),
                        pltpu.VMEM((1, out_dim), f32),
                        pltpu.VMEM((1, out_dim), f32)],
        compiler_params=pltpu.CompilerParams(
            dimension_semantics=("arbitrary", "arbitrary"),
            vmem_limit_bytes=_VMEM_LIMIT),
    )(x2, wt, gamma.reshape(1, out_dim).astype(f32),
      beta.reshape(1, out_dim).astype(f32))

    return out.reshape(B, N, out_dim)


def kernel(x, weight, bias, gamma, beta):
    del bias
    return _run(x, weight, gamma, beta)
